# Initial kernel scaffold; baseline (speedup 1.0000x reference)
import jax
import jax.numpy as jnp
from jax.experimental import pallas as pl


def kernel(x_user, x_image, x_text, edge_index_image_user, edge_index_text_user, W_user, b_user, Wl_img, bl_img, Wr_img, Wl_txt, bl_txt, Wr_txt):
    raise NotImplementedError("write your pallas kernel here")


# speedup vs baseline: 8.3156x; 8.3156x over previous
import functools

import jax
import jax.numpy as jnp
from jax import lax
from jax.experimental import pallas as pl
from jax.experimental.pallas import tpu as pltpu
from jax.experimental.pallas import tpu_sc as plsc

N_USER = 10000
N_SRC = 10000
E = 320000
D = 128

_LANES = 16
_NTILES = 16
_ROWS_PER_IDX = 64
_GROUP = 8
_GBUF = 4
_NROWS = E // _ROWS_PER_IDX
_NGROUPS = _NROWS // _GROUP
_ROWS_OUT = 624
_ROWS_TAIL = N_USER - _ROWS_OUT * _NTILES


def _sc_segment_sums(x_image, x_text, src_img, dst_img, src_txt, dst_txt,
                     zrows, zcnt):
  mesh = plsc.VectorSubcoreMesh(core_axis_name="c", subcore_axis_name="s")

  @functools.partial(
      pl.kernel,
      out_type=(
          jax.ShapeDtypeStruct((2, N_USER, D), jnp.float32),
          jax.ShapeDtypeStruct((2, N_USER), jnp.float32),
      ),
      mesh=mesh,
      scratch_types=[
          pltpu.VMEM((_GROUP, _ROWS_PER_IDX), jnp.int32),
          pltpu.VMEM((_GROUP, _ROWS_PER_IDX), jnp.int32),
          pltpu.VMEM((_GBUF, _ROWS_PER_IDX, D), jnp.float32),
          pltpu.VMEM((_ROWS_PER_IDX,), jnp.float32),
          pltpu.VMEM_SHARED((N_USER, D), jnp.float32),
          pltpu.VMEM_SHARED((N_USER,), jnp.float32),
          pltpu.SemaphoreType.DMA,
      ],
  )
  def seg_kernel(x_img_hbm, x_txt_hbm, s_img_hbm, d_img_hbm, s_txt_hbm,
                 d_txt_hbm, zrows_hbm, zcnt_hbm, out_s_hbm, out_c_hbm,
                 sidx, didx, rows, ones, acc, cnt, gsem):
    cid = lax.axis_index("c")
    sid = lax.axis_index("s")

    pltpu.sync_copy(zrows_hbm.at[pl.ds(0, _ROWS_OUT), :],
                    acc.at[pl.ds(sid * _ROWS_OUT, _ROWS_OUT), :])
    @pl.when(sid == 0)
    def _():
      pltpu.sync_copy(zcnt_hbm, cnt)
      pltpu.sync_copy(zrows_hbm.at[pl.ds(0, _ROWS_TAIL), :],
                      acc.at[pl.ds(_ROWS_OUT * _NTILES, _ROWS_TAIL), :])
    for i in range(_ROWS_PER_IDX // _LANES):
      ones[pl.ds(i * _LANES, _LANES)] = jnp.ones((_LANES,), jnp.float32)
    plsc.subcore_barrier()

    def main_loop(x_tbl, src2, dst2):
      n_g = jnp.where(sid < (_NGROUPS % _NTILES), _NGROUPS // _NTILES + 1,
                      _NGROUPS // _NTILES)

      def group_body(i, _):
        g = sid + i * _NTILES
        rbase = g * _GROUP
        pltpu.sync_copy(src2.at[pl.ds(rbase, _GROUP), :], sidx)
        pltpu.sync_copy(dst2.at[pl.ds(rbase, _GROUP), :], didx)
        for h in range(_GROUP // _GBUF):
          waits = []
          for j in range(_GBUF):
            waits.append(pltpu.async_copy(
                x_tbl.at[sidx.at[h * _GBUF + j]], rows.at[j], gsem))
          for w in waits:
            w.wait()
          for j in range(_GBUF):
            pltpu.sync_copy(rows.at[j], acc.at[didx.at[h * _GBUF + j]],
                            add=True)
            pltpu.sync_copy(ones, cnt.at[didx.at[h * _GBUF + j]], add=True)
        return 0

      lax.fori_loop(0, n_g, group_body, 0)

    @pl.when(cid == 0)
    def _():
      main_loop(x_img_hbm, s_img_hbm, d_img_hbm)
    @pl.when(cid == 1)
    def _():
      main_loop(x_txt_hbm, s_txt_hbm, d_txt_hbm)

    plsc.subcore_barrier()

    def writeout(rel):
      pltpu.sync_copy(acc.at[pl.ds(sid * _ROWS_OUT, _ROWS_OUT), :],
                      out_s_hbm.at[rel, pl.ds(sid * _ROWS_OUT, _ROWS_OUT), :])
      @pl.when(sid == 0)
      def _():
        pltpu.sync_copy(cnt, out_c_hbm.at[rel])
        pltpu.sync_copy(
            acc.at[pl.ds(_ROWS_OUT * _NTILES, _ROWS_TAIL), :],
            out_s_hbm.at[rel, pl.ds(_ROWS_OUT * _NTILES, _ROWS_TAIL), :])

    @pl.when(cid == 0)
    def _():
      writeout(0)
    @pl.when(cid == 1)
    def _():
      writeout(1)

  return seg_kernel(x_image, x_text, src_img, dst_img, src_txt, dst_txt,
                    zrows, zcnt)


def _tc_combine(x_user, sums, cnts3, W_user, b_user2, Wl_img, Wl_txt,
                Wr_img, Wr_txt, bl_img2, bl_txt2):
  blk = 1000
  grid = (N_USER // blk,)

  def dotT(a, b):
    return lax.dot_general(a, b, (((1,), (1,)), ((), ())),
                           preferred_element_type=jnp.float32)

  def body(xu_ref, si_ref, st_ref, ci_ref, ct_ref, Wu_ref, bu_ref,
           Wli_ref, Wlt_ref, Wri_ref, Wrt_ref, bli_ref, blt_ref, out_ref):
    xu = dotT(xu_ref[...], Wu_ref[...]) + bu_ref[...]
    ci = jnp.maximum(ci_ref[0, :, :], 1.0)
    ct = jnp.maximum(ct_ref[0, :, :], 1.0)
    mi = si_ref[0] / ci
    mt = st_ref[0] / ct
    out = (dotT(mi, Wli_ref[...]) + dotT(mt, Wlt_ref[...])
           + dotT(xu, Wri_ref[...]) + dotT(xu, Wrt_ref[...])
           + bli_ref[...] + blt_ref[...])
    out_ref[...] = jnp.maximum(out, 0.0)

  full2 = pl.BlockSpec((128, 128), lambda i: (0, 0))
  bias2 = pl.BlockSpec((1, 128), lambda i: (0, 0))
  return pl.pallas_call(
      body,
      grid=grid,
      in_specs=[
          pl.BlockSpec((blk, D), lambda i: (i, 0)),
          pl.BlockSpec((1, blk, D), lambda i: (0, i, 0)),
          pl.BlockSpec((1, blk, D), lambda i: (1, i, 0)),
          pl.BlockSpec((1, blk, 1), lambda i: (0, i, 0)),
          pl.BlockSpec((1, blk, 1), lambda i: (1, i, 0)),
          full2, bias2, full2, full2, full2, full2, bias2, bias2,
      ],
      out_specs=pl.BlockSpec((blk, D), lambda i: (i, 0)),
      out_shape=jax.ShapeDtypeStruct((N_USER, D), jnp.float32),
  )(x_user, sums, sums, cnts3, cnts3, W_user, b_user2, Wl_img, Wl_txt,
    Wr_img, Wr_txt, bl_img2, bl_txt2)


def kernel(x_user, x_image, x_text, edge_index_image_user,
           edge_index_text_user, W_user, b_user, Wl_img, bl_img, Wr_img,
           Wl_txt, bl_txt, Wr_txt):
  src_img = edge_index_image_user[0].reshape(_NROWS, _ROWS_PER_IDX)
  dst_img = edge_index_image_user[1].reshape(_NROWS, _ROWS_PER_IDX)
  src_txt = edge_index_text_user[0].reshape(_NROWS, _ROWS_PER_IDX)
  dst_txt = edge_index_text_user[1].reshape(_NROWS, _ROWS_PER_IDX)
  zrows = jnp.zeros((_ROWS_OUT, D), jnp.float32)
  zcnt = jnp.zeros((N_USER,), jnp.float32)

  sums, cnts = _sc_segment_sums(x_image, x_text, src_img, dst_img,
                                src_txt, dst_txt, zrows, zcnt)

  return _tc_combine(
      x_user, sums, cnts.reshape(2, N_USER, 1), W_user,
      b_user.reshape(1, D), Wl_img, Wl_txt, Wr_img, Wr_txt,
      bl_img.reshape(1, D), bl_txt.reshape(1, D))
